# baseline (speedup 1.0000x reference)
import functools

import jax
import jax.numpy as jnp
from jax.experimental import pallas as pl
from jax.experimental.pallas import tpu as pltpu

_VMEM_LIMIT_BYTES = 64 * 1024 * 1024


def _fpn_level_kernel(H, W, HWc, RC, has_skip, emit_lat, *refs):
    i = 0
    x_ref = refs[i]; i += 1
    w1_ref = refs[i]; i += 1
    b1_ref = refs[i]; i += 1
    if has_skip:
        skip_ref = refs[i]; i += 1
    wd_ref = refs[i]; i += 1
    b3_ref = refs[i]; i += 1
    if emit_lat:
        lat_ref = refs[i]; i += 1
    out_ref = refs[i]; i += 1
    lat_s = refs[i]; i += 1
    p_ref = refs[i]

    C = w1_ref.shape[1]
    HW = H * W
    MT = HW // HWc
    CH = RC * W
    s = pl.program_id(1)

    @pl.when(s < MT)
    def _lateral():
        x = x_ref[...].astype(jnp.bfloat16)
        latc = jax.lax.dot_general(x, w1_ref[...], (((0,), (0,)), ((), ())),
                                   preferred_element_type=jnp.float32)
        lat_s[pl.ds(s * HWc, HWc), :] = latc + b1_ref[...]

    @pl.when(s == MT)
    def _finalize():
        lat = lat_s[...]
        if has_skip:
            H2, W2 = H // 2, W // 2
            prev = skip_ref[...]
            t = jnp.broadcast_to(prev.reshape(H2 * W2, 1, C), (H2 * W2, 2, C))
            t = t.reshape(H2 * W, C)
            t = jnp.broadcast_to(t.reshape(H2, 1, W, C), (H2, 2, W, C))
            lat = lat + t.reshape(HW, C)
        if emit_lat:
            lat_ref[...] = lat

        latb = lat.astype(jnp.bfloat16).reshape(H, W, C)
        zrow = jnp.zeros((1, W, 3 * C), jnp.bfloat16)
        zcol = jnp.zeros((H + 2, 8, C), jnp.bfloat16)
        p_ref[0:1, :, :] = zrow
        p_ref[H + 1:H + 2, :, :] = zrow
        p_ref[:, 0:8, 0:C] = zcol
        p_ref[:, W - 8:W, 2 * C:3 * C] = zcol
        p_ref[1:H + 1, 1:W, 0:C] = latb[:, 0:W - 1, :]
        p_ref[1:H + 1, :, C:2 * C] = latb
        p_ref[1:H + 1, 0:W - 1, 2 * C:3 * C] = latb[:, 1:W, :]

    @pl.when(s >= MT)
    def _conv_chunk():
        c = s - MT
        acc = None
        for dy in range(3):
            patch = p_ref[pl.ds(c * RC + dy, RC), :, :].reshape(CH, 3 * C)
            d = jnp.dot(patch, wd_ref[dy], preferred_element_type=jnp.float32)
            acc = d if acc is None else acc + d
        out_ref[...] = jnp.transpose(acc + b3_ref[...])


def _fpn_level(x, H, W, w1, b1, skip, wd, b3, emit_lat):
    N, Cin, HW = x.shape
    C = w1.shape[1]
    HWc = min(HW, max(128, (2 * 1024 * 1024) // (Cin * 4)))
    MT = HW // HWc
    RC = max(1, min(H, 512 // W))
    CH = RC * W
    NC = H // RC
    S = MT + NC

    in_specs = [
        pl.BlockSpec((None, Cin, HWc), lambda n, s: (n, 0, jnp.minimum(s, MT - 1))),
        pl.BlockSpec((Cin, C), lambda n, s: (0, 0)),
        pl.BlockSpec((1, C), lambda n, s: (0, 0)),
    ]
    args = [x, w1, b1]
    if skip is not None:
        in_specs.append(pl.BlockSpec((None, HW // 4, C), lambda n, s: (n, 0, 0)))
        args.append(skip)
    in_specs += [
        pl.BlockSpec((3, 3 * C, C), lambda n, s: (0, 0, 0)),
        pl.BlockSpec((1, C), lambda n, s: (0, 0)),
    ]
    args += [wd, b3]

    out_shape = [jax.ShapeDtypeStruct((N, C, HW), jnp.float32)]
    out_specs = [pl.BlockSpec(
        (None, C, CH), lambda n, s: (n, 0, jnp.clip(s - MT, 0, NC - 1)))]
    if emit_lat:
        out_shape.append(jax.ShapeDtypeStruct((N, HW, C), jnp.float32))
        out_specs.append(pl.BlockSpec((None, HW, C), lambda n, s: (n, 0, 0)))
        out_shape = out_shape[::-1]
        out_specs = out_specs[::-1]

    body = functools.partial(_fpn_level_kernel, H, W, HWc, RC,
                             skip is not None, emit_lat)
    res = pl.pallas_call(
        body,
        grid=(N, S),
        in_specs=in_specs,
        out_specs=tuple(out_specs),
        out_shape=tuple(out_shape),
        scratch_shapes=[pltpu.VMEM((HW, C), jnp.float32),
                        pltpu.VMEM((H + 2, W, 3 * C), jnp.bfloat16)],
        compiler_params=pltpu.CompilerParams(
            dimension_semantics=("parallel", "arbitrary"),
            vmem_limit_bytes=_VMEM_LIMIT_BYTES,
        ),
    )(*args)
    if emit_lat:
        return res[0], res[1]
    return None, res[0]


def _dx_concat(w9):
    w = w9.astype(jnp.bfloat16)
    return jnp.stack([jnp.concatenate([w[3 * dy], w[3 * dy + 1], w[3 * dy + 2]],
                                      axis=0) for dy in range(3)])


def kernel(c3, c4, c5,
           p5_1_w, p5_1_b, p5_2_w, p5_2_b,
           p4_1_w, p4_1_b, p4_2_w, p4_2_b,
           p3_1_w, p3_1_b, p3_2_w, p3_2_b):
    N = c3.shape[0]
    bf = jnp.bfloat16

    def to_cm(x):
        n, c, h, w = x.shape
        return x.reshape(n, c, h * w)

    x5, x4, x3 = to_cm(c5), to_cm(c4), to_cm(c3)
    h5, w5 = c5.shape[2], c5.shape[3]
    h4, w4 = c4.shape[2], c4.shape[3]
    h3, w3_ = c3.shape[2], c3.shape[3]

    p5_lat, p5_out = _fpn_level(x5, h5, w5, p5_1_w.astype(bf), p5_1_b,
                                None, _dx_concat(p5_2_w), p5_2_b, emit_lat=True)
    p4_lat, p4_out = _fpn_level(x4, h4, w4, p4_1_w.astype(bf), p4_1_b,
                                p5_lat, _dx_concat(p4_2_w), p4_2_b, emit_lat=True)
    _, p3_out = _fpn_level(x3, h3, w3_, p3_1_w.astype(bf), p3_1_b,
                           p4_lat, _dx_concat(p3_2_w), p3_2_b, emit_lat=False)

    def to_nchw(o, h, w):
        return o.reshape(N, o.shape[1], h, w)

    return [to_nchw(p3_out, h3, w3_), to_nchw(p4_out, h4, w4),
            to_nchw(p5_out, h5, w5)]


# speedup vs baseline: 1.1998x; 1.1998x over previous
import functools

import jax
import jax.numpy as jnp
from jax.experimental import pallas as pl
from jax.experimental.pallas import tpu as pltpu

_VMEM_LIMIT_BYTES = 100 * 1024 * 1024


def _upsample2x(prev, H2, W2, C):
    t = jnp.broadcast_to(prev.reshape(H2 * W2, 1, C), (H2 * W2, 2, C))
    t = t.reshape(H2 * 2 * W2, C)
    t = jnp.broadcast_to(t.reshape(H2, 1, 2 * W2, C), (H2, 2, 2 * W2, C))
    return t.reshape(4 * H2 * W2, C)


def _level_body(H, W, x_ref, w1_ref, b1_ref, prev_s, wd_ref, b3_ref,
                lat_s, out_ref, p_ref):
    C = w1_ref.shape[1]
    HW = H * W

    x = x_ref[...].astype(jnp.bfloat16)
    lat = jax.lax.dot_general(x, w1_ref[...], (((0,), (0,)), ((), ())),
                              preferred_element_type=jnp.float32)
    lat = lat + b1_ref[...]
    if prev_s is not None:
        lat = lat + _upsample2x(prev_s[...], H // 2, W // 2, C)
    if lat_s is not None:
        lat_s[...] = lat

    latb = lat.astype(jnp.bfloat16).reshape(H, W, C)
    zrow = jnp.zeros((1, W, 3 * C), jnp.bfloat16)
    zcol = jnp.zeros((H + 2, 8, C), jnp.bfloat16)
    p_ref[0:1, :, :] = zrow
    p_ref[H + 1:H + 2, :, :] = zrow
    p_ref[:, 0:8, 0:C] = zcol
    p_ref[:, W - 8:W, 2 * C:3 * C] = zcol
    p_ref[1:H + 1, 1:W, 0:C] = latb[:, 0:W - 1, :]
    p_ref[1:H + 1, :, C:2 * C] = latb
    p_ref[1:H + 1, 0:W - 1, 2 * C:3 * C] = latb[:, 1:W, :]

    RC = max(1, min(H, 512 // W))
    CH = RC * W
    for mc in range(H // RC):
        r0 = mc * RC
        acc = None
        for dy in range(3):
            patch = p_ref[r0 + dy:r0 + dy + RC, :, :].reshape(CH, 3 * C)
            d = jnp.dot(patch, wd_ref[dy], preferred_element_type=jnp.float32)
            acc = d if acc is None else acc + d
        out_ref[:, mc * CH:(mc + 1) * CH] = jnp.transpose(acc + b3_ref[...])


def _fpn_kernel(dims,
                x5_ref, w15_ref, b15_ref, wd5_ref, b35_ref,
                x4_ref, w14_ref, b14_ref, wd4_ref, b34_ref,
                x3_ref, w13_ref, b13_ref, wd3_ref, b33_ref,
                out3_ref, out4_ref, out5_ref,
                lat5_s, lat4_s, p5_s, p4_s, p3_s):
    (h5, w5), (h4, w4), (h3, w3) = dims
    lvl = pl.program_id(1)

    @pl.when(lvl == 0)
    def _p5():
        _level_body(h5, w5, x5_ref, w15_ref, b15_ref, None, wd5_ref, b35_ref,
                    lat5_s, out5_ref, p5_s)

    @pl.when(lvl == 1)
    def _p4():
        _level_body(h4, w4, x4_ref, w14_ref, b14_ref, lat5_s, wd4_ref, b34_ref,
                    lat4_s, out4_ref, p4_s)

    @pl.when(lvl == 2)
    def _p3():
        _level_body(h3, w3, x3_ref, w13_ref, b13_ref, lat4_s, wd3_ref, b33_ref,
                    None, out3_ref, p3_s)


def _dx_concat(w9):
    w = w9.astype(jnp.bfloat16)
    return jnp.stack([jnp.concatenate([w[3 * dy], w[3 * dy + 1], w[3 * dy + 2]],
                                      axis=0) for dy in range(3)])


def kernel(c3, c4, c5,
           p5_1_w, p5_1_b, p5_2_w, p5_2_b,
           p4_1_w, p4_1_b, p4_2_w, p4_2_b,
           p3_1_w, p3_1_b, p3_2_w, p3_2_b):
    N = c3.shape[0]
    bf = jnp.bfloat16
    C = p5_1_w.shape[1]

    def to_cm(x):
        n, c, h, w = x.shape
        return x.reshape(n, c, h * w)

    x5, x4, x3 = to_cm(c5), to_cm(c4), to_cm(c3)
    h5, w5 = c5.shape[2], c5.shape[3]
    h4, w4 = c4.shape[2], c4.shape[3]
    h3, w3_ = c3.shape[2], c3.shape[3]
    dims = ((h5, w5), (h4, w4), (h3, w3_))

    def full(a):
        shape = a.shape
        return pl.BlockSpec(shape, lambda n, s: (0,) * len(shape))

    def batched(a):
        shape = a.shape[1:]
        return pl.BlockSpec((None,) + shape, lambda n, s: (n,) + (0,) * len(shape))

    args = [
        x5, p5_1_w.astype(bf), p5_1_b, _dx_concat(p5_2_w), p5_2_b,
        x4, p4_1_w.astype(bf), p4_1_b, _dx_concat(p4_2_w), p4_2_b,
        x3, p3_1_w.astype(bf), p3_1_b, _dx_concat(p3_2_w), p3_2_b,
    ]
    in_specs = []
    for k, a in enumerate(args):
        in_specs.append(batched(a) if k % 5 == 0 else full(a))

    out_shape = (jax.ShapeDtypeStruct((N, C, h3 * w3_), jnp.float32),
                 jax.ShapeDtypeStruct((N, C, h4 * w4), jnp.float32),
                 jax.ShapeDtypeStruct((N, C, h5 * w5), jnp.float32))
    out_specs = (pl.BlockSpec((None, C, h3 * w3_), lambda n, s: (n, 0, 0)),
                 pl.BlockSpec((None, C, h4 * w4), lambda n, s: (n, 0, 0)),
                 pl.BlockSpec((None, C, h5 * w5), lambda n, s: (n, 0, 0)))

    res = pl.pallas_call(
        functools.partial(_fpn_kernel, dims),
        grid=(N, 3),
        in_specs=in_specs,
        out_specs=out_specs,
        out_shape=out_shape,
        scratch_shapes=[
            pltpu.VMEM((h5 * w5, C), jnp.float32),
            pltpu.VMEM((h4 * w4, C), jnp.float32),
            pltpu.VMEM((h5 + 2, w5, 3 * C), jnp.bfloat16),
            pltpu.VMEM((h4 + 2, w4, 3 * C), jnp.bfloat16),
            pltpu.VMEM((h3 + 2, w3_, 3 * C), jnp.bfloat16),
        ],
        compiler_params=pltpu.CompilerParams(
            dimension_semantics=("parallel", "arbitrary"),
            vmem_limit_bytes=_VMEM_LIMIT_BYTES,
        ),
    )(*args)

    p3_out, p4_out, p5_out = res
    return [p3_out.reshape(N, C, h3, w3_), p4_out.reshape(N, C, h4, w4),
            p5_out.reshape(N, C, h5, w5)]


# speedup vs baseline: 1.3530x; 1.1277x over previous
import functools

import jax
import jax.numpy as jnp
from jax.experimental import pallas as pl
from jax.experimental.pallas import tpu as pltpu

_VMEM_LIMIT_BYTES = 100 * 1024 * 1024


def _upsample2x(prev, H2, W2, C):
    t = jnp.broadcast_to(prev.reshape(H2 * W2, 1, C), (H2 * W2, 2, C))
    t = t.reshape(H2 * 2 * W2, C)
    t = jnp.broadcast_to(t.reshape(H2, 1, 2 * W2, C), (H2, 2, 2 * W2, C))
    return t.reshape(4 * H2 * W2, C)


def _level_body(H, W, x_ref, w1_ref, b1_ref, prev_s, w9_ref, b3_ref,
                lat_s, out_ref, p_ref):
    C = w1_ref.shape[1]
    HW = H * W

    x = x_ref[...].astype(jnp.bfloat16)
    lat = jax.lax.dot_general(x, w1_ref[...].astype(jnp.bfloat16),
                              (((0,), (0,)), ((), ())),
                              preferred_element_type=jnp.float32)
    lat = lat + b1_ref[...]
    if prev_s is not None:
        lat = lat + _upsample2x(prev_s[...], H // 2, W // 2, C)
    if lat_s is not None:
        lat_s[...] = lat

    latb = lat.astype(jnp.bfloat16).reshape(H, W, C)
    zrow = jnp.zeros((1, W, 3 * C), jnp.bfloat16)
    zcol = jnp.zeros((H + 2, 8, C), jnp.bfloat16)
    p_ref[0:1, :, :] = zrow
    p_ref[H + 1:H + 2, :, :] = zrow
    p_ref[:, 0:8, 0:C] = zcol
    p_ref[:, W - 8:W, 2 * C:3 * C] = zcol
    p_ref[1:H + 1, 1:W, 0:C] = latb[:, 0:W - 1, :]
    p_ref[1:H + 1, :, C:2 * C] = latb
    p_ref[1:H + 1, 0:W - 1, 2 * C:3 * C] = latb[:, 1:W, :]

    wd = [jnp.concatenate([w9_ref[3 * dy + b].astype(jnp.bfloat16)
                           for b in range(3)], axis=0) for dy in range(3)]
    RC = max(1, min(H, 512 // W))
    CH = RC * W
    for mc in range(H // RC):
        r0 = mc * RC
        acc = None
        for dy in range(3):
            patch = p_ref[r0 + dy:r0 + dy + RC, :, :].reshape(CH, 3 * C)
            d = jnp.dot(patch, wd[dy], preferred_element_type=jnp.float32)
            acc = d if acc is None else acc + d
        out_ref[:, mc * CH:(mc + 1) * CH] = jnp.transpose(acc + b3_ref[...])


def _fpn_kernel(dims,
                x5_ref, w15_ref, b15_ref, wd5_ref, b35_ref,
                x4_ref, w14_ref, b14_ref, wd4_ref, b34_ref,
                x3_ref, w13_ref, b13_ref, wd3_ref, b33_ref,
                out3_ref, out4_ref, out5_ref,
                lat5_s, lat4_s, p5_s, p4_s, p3_s):
    (h5, w5), (h4, w4), (h3, w3) = dims
    lvl = pl.program_id(1)

    @pl.when(lvl == 0)
    def _p5():
        _level_body(h5, w5, x5_ref, w15_ref, b15_ref, None, wd5_ref, b35_ref,
                    lat5_s, out5_ref, p5_s)

    @pl.when(lvl == 1)
    def _p4():
        _level_body(h4, w4, x4_ref, w14_ref, b14_ref, lat5_s, wd4_ref, b34_ref,
                    lat4_s, out4_ref, p4_s)

    @pl.when(lvl == 2)
    def _p3():
        _level_body(h3, w3, x3_ref, w13_ref, b13_ref, lat4_s, wd3_ref, b33_ref,
                    None, out3_ref, p3_s)


def kernel(c3, c4, c5,
           p5_1_w, p5_1_b, p5_2_w, p5_2_b,
           p4_1_w, p4_1_b, p4_2_w, p4_2_b,
           p3_1_w, p3_1_b, p3_2_w, p3_2_b):
    N = c3.shape[0]
    bf = jnp.bfloat16
    C = p5_1_w.shape[1]

    def to_cm(x):
        n, c, h, w = x.shape
        return x.reshape(n, c, h * w)

    x5, x4, x3 = to_cm(c5), to_cm(c4), to_cm(c3)
    h5, w5 = c5.shape[2], c5.shape[3]
    h4, w4 = c4.shape[2], c4.shape[3]
    h3, w3_ = c3.shape[2], c3.shape[3]
    dims = ((h5, w5), (h4, w4), (h3, w3_))

    def full(a):
        shape = a.shape
        return pl.BlockSpec(shape, lambda n, s: (0,) * len(shape))

    def batched(a):
        shape = a.shape[1:]
        return pl.BlockSpec((None,) + shape, lambda n, s: (n,) + (0,) * len(shape))

    args = [
        x5, p5_1_w, p5_1_b, p5_2_w, p5_2_b,
        x4, p4_1_w, p4_1_b, p4_2_w, p4_2_b,
        x3, p3_1_w, p3_1_b, p3_2_w, p3_2_b,
    ]
    in_specs = []
    for k, a in enumerate(args):
        in_specs.append(batched(a) if k % 5 == 0 else full(a))

    out_shape = (jax.ShapeDtypeStruct((N, C, h3 * w3_), jnp.float32),
                 jax.ShapeDtypeStruct((N, C, h4 * w4), jnp.float32),
                 jax.ShapeDtypeStruct((N, C, h5 * w5), jnp.float32))
    out_specs = (pl.BlockSpec((None, C, h3 * w3_), lambda n, s: (n, 0, 0)),
                 pl.BlockSpec((None, C, h4 * w4), lambda n, s: (n, 0, 0)),
                 pl.BlockSpec((None, C, h5 * w5), lambda n, s: (n, 0, 0)))

    res = pl.pallas_call(
        functools.partial(_fpn_kernel, dims),
        grid=(N, 3),
        in_specs=in_specs,
        out_specs=out_specs,
        out_shape=out_shape,
        scratch_shapes=[
            pltpu.VMEM((h5 * w5, C), jnp.float32),
            pltpu.VMEM((h4 * w4, C), jnp.float32),
            pltpu.VMEM((h5 + 2, w5, 3 * C), jnp.bfloat16),
            pltpu.VMEM((h4 + 2, w4, 3 * C), jnp.bfloat16),
            pltpu.VMEM((h3 + 2, w3_, 3 * C), jnp.bfloat16),
        ],
        compiler_params=pltpu.CompilerParams(
            dimension_semantics=("parallel", "arbitrary"),
            vmem_limit_bytes=_VMEM_LIMIT_BYTES,
        ),
    )(*args)

    p3_out, p4_out, p5_out = res
    return [p3_out.reshape(N, C, h3, w3_), p4_out.reshape(N, C, h4, w4),
            p5_out.reshape(N, C, h5, w5)]
